# baseline (speedup 1.0000x reference)
import functools

import jax
import jax.numpy as jnp
from jax import lax
from jax.experimental import pallas as pl
from jax.experimental.pallas import tpu as pltpu
from jax.experimental.pallas import tpu_sc as plsc

N_NODES = 10000
D = 128
NUM_ET = 3
E_PER = 106667
L = 2

NW = 32
NPAD = 10240
ROWS_PER_W = NPAD // NW
ROWS_PER_TILE = NPAD // 16

CH = 128
NCH = 28
EW = NCH * CH
EPAD = NW * EW

GCH = 80
NGCH = ROWS_PER_W // GCH

_mesh = plsc.VectorSubcoreMesh(core_axis_name="c", subcore_axis_name="s")


@functools.partial(
    pl.kernel,
    out_type=jax.ShapeDtypeStruct((NPAD, D), jnp.float32),
    mesh=_mesh,
    scratch_types=[
        pltpu.VMEM((GCH,), jnp.int32),
        pltpu.VMEM((GCH, D), jnp.float32),
        pltpu.SemaphoreType.DMA,
    ],
)
def _emb_gather(table_hbm, ids_hbm, out_hbm, idx_v, rows_v, sem):
    c = lax.axis_index("c")
    s = lax.axis_index("s")
    wid = s * 2 + c
    base = wid * ROWS_PER_W
    for k in range(NGCH):
        off = base + k * GCH
        pltpu.sync_copy(ids_hbm.at[pl.ds(off, GCH)], idx_v)
        pltpu.async_copy(table_hbm.at[idx_v], rows_v, sem).wait()
        pltpu.sync_copy(rows_v, out_hbm.at[pl.ds(off, GCH)])


@functools.partial(
    pl.kernel,
    out_type=jax.ShapeDtypeStruct((2, NUM_ET, NPAD, D), jnp.float32),
    mesh=_mesh,
    scratch_types=[
        pltpu.VMEM_SHARED((NPAD, D), jnp.float32),
        pltpu.VMEM((NCH + 1, CH), jnp.int32),
        pltpu.VMEM((NCH + 1, CH), jnp.int32),
        pltpu.VMEM((CH,), jnp.int32),
        pltpu.VMEM((CH,), jnp.int32),
        pltpu.VMEM((CH,), jnp.int32),
        pltpu.VMEM((CH,), jnp.int32),
        pltpu.VMEM((CH, D), jnp.float32),
        pltpu.VMEM((CH, D), jnp.float32),
        pltpu.SemaphoreType.DMA,
        pltpu.SemaphoreType.DMA,
    ],
)
def _seg_sums(h_hbm, src0, dst0, src1, dst1, src2, dst2, zeros_hbm, out_hbm,
              acc_sh, src_all, dst_all, srcA, dstA, srcB, dstB,
              rowsA, rowsB, gsemA, gsemB):
    c = lax.axis_index("c")
    s = lax.axis_index("s")
    wid = s * 2 + c
    tile_lo = s * ROWS_PER_TILE

    def copy_idx(dst_buf, src_ref, k):
        for j in range(CH // 16):
            dst_buf[pl.ds(j * 16, 16)] = src_ref[k, pl.ds(j * 16, 16)]

    def wait(sem, rows):
        pltpu.make_async_copy(h_hbm.at[pl.ds(0, CH)], rows, sem).wait()

    edge_lists = ((src0, dst0), (src1, dst1), (src2, dst2))
    for t in range(NUM_ET):
        src_hbm, dst_hbm = edge_lists[t]
        pltpu.sync_copy(src_hbm.at[wid], src_all)
        pltpu.sync_copy(dst_hbm.at[wid], dst_all)
        pltpu.sync_copy(zeros_hbm, acc_sh.at[pl.ds(tile_lo, ROWS_PER_TILE)])
        plsc.subcore_barrier()

        copy_idx(srcA, src_all, 0)
        copy_idx(dstA, dst_all, 0)
        pltpu.async_copy(h_hbm.at[srcA], rowsA, gsemA)

        def body(g, carry):
            copy_idx(srcB, src_all, 2 * g + 1)
            copy_idx(dstB, dst_all, 2 * g + 1)
            wait(gsemA, rowsA)
            pltpu.async_copy(h_hbm.at[srcB], rowsB, gsemB)
            pltpu.sync_copy(rowsA, acc_sh.at[dstA], add=True)
            copy_idx(srcA, src_all, 2 * g + 2)
            copy_idx(dstA, dst_all, 2 * g + 2)
            wait(gsemB, rowsB)
            pltpu.async_copy(h_hbm.at[srcA], rowsA, gsemA)
            pltpu.sync_copy(rowsB, acc_sh.at[dstB], add=True)
            return carry

        lax.fori_loop(0, NCH // 2, body, 0)
        wait(gsemA, rowsA)
        plsc.subcore_barrier()
        pltpu.sync_copy(
            acc_sh.at[pl.ds(tile_lo, ROWS_PER_TILE)],
            out_hbm.at[c, t, pl.ds(tile_lo, ROWS_PER_TILE)],
        )


BM = 512


def _layer_body(h_ref, s_ref, wself_ref, wmsg_ref, out_ref):
    acc = jnp.dot(h_ref[...], wself_ref[...], preferred_element_type=jnp.float32)
    for t in range(NUM_ET):
        st = s_ref[0, t] + s_ref[1, t]
        acc += jnp.dot(st, wmsg_ref[t], preferred_element_type=jnp.float32)
    out_ref[...] = jnp.maximum(acc, 0.0)


def _tc_layer(h, S, wself, wmsg):
    return pl.pallas_call(
        _layer_body,
        grid=(NPAD // BM,),
        in_specs=[
            pl.BlockSpec((BM, D), lambda i: (i, 0)),
            pl.BlockSpec((2, NUM_ET, BM, D), lambda i: (0, 0, i, 0)),
            pl.BlockSpec((D, D), lambda i: (0, 0)),
            pl.BlockSpec((NUM_ET, D, D), lambda i: (0, 0, 0)),
        ],
        out_specs=pl.BlockSpec((BM, D), lambda i: (i, 0)),
        out_shape=jax.ShapeDtypeStruct((NPAD, D), jnp.float32),
    )(h, S, wself, wmsg)


def kernel(node_label_ids, adjacency_list_0, adjacency_list_1, adjacency_list_2,
           node_to_graph_map, num_graphs, emb_table, W_msg, W_self):
    ids = jnp.zeros((NPAD,), jnp.int32).at[:N_NODES].set(
        node_label_ids.astype(jnp.int32))
    srcs, dsts = [], []
    dummy_src = jnp.zeros((NW, 1, CH), jnp.int32)
    dummy_dst = jnp.full((NW, 1, CH), NPAD - 1, jnp.int32)
    for a in (adjacency_list_0, adjacency_list_1, adjacency_list_2):
        a = a.astype(jnp.int32)
        srcs.append(jnp.concatenate(
            [jnp.zeros((EPAD,), jnp.int32).at[:E_PER].set(a[:, 0])
             .reshape(NW, NCH, CH), dummy_src], axis=1))
        dsts.append(jnp.concatenate(
            [jnp.full((EPAD,), NPAD - 1, jnp.int32).at[:E_PER].set(a[:, 1])
             .reshape(NW, NCH, CH), dummy_dst], axis=1))
    zeros = jnp.zeros((ROWS_PER_TILE, D), jnp.float32)

    h = _emb_gather(emb_table, ids)
    for layer in range(L):
        S = _seg_sums(h, srcs[0], dsts[0], srcs[1], dsts[1], srcs[2], dsts[2],
                      zeros)
        h = _tc_layer(h, S, W_self[layer], W_msg[layer])
    return h[:N_NODES]


# speedup vs baseline: 1.4197x; 1.4197x over previous
import functools

import jax
import jax.numpy as jnp
from jax import lax
from jax.experimental import pallas as pl
from jax.experimental.pallas import tpu as pltpu
from jax.experimental.pallas import tpu_sc as plsc

N_NODES = 10000
D = 128
NUM_ET = 3
E_PER = 106667
L = 2

NW = 32
NPAD = 10240
ROWS_PER_W = NPAD // NW
ROWS_PER_TILE = NPAD // 16

CH = 128
NCH = 28
EW = NCH * CH
EPAD = NW * EW
NB = 2

GCH = 80
NGCH = ROWS_PER_W // GCH

_mesh = plsc.VectorSubcoreMesh(core_axis_name="c", subcore_axis_name="s")


@functools.partial(
    pl.kernel,
    out_type=jax.ShapeDtypeStruct((NPAD, D), jnp.float32),
    mesh=_mesh,
    scratch_types=[
        pltpu.VMEM((GCH,), jnp.int32),
        pltpu.VMEM((GCH, D), jnp.float32),
        pltpu.SemaphoreType.DMA,
    ],
)
def _emb_gather(table_hbm, ids_hbm, out_hbm, idx_v, rows_v, sem):
    c = lax.axis_index("c")
    s = lax.axis_index("s")
    wid = s * 2 + c
    base = wid * ROWS_PER_W
    for k in range(NGCH):
        off = base + k * GCH
        pltpu.sync_copy(ids_hbm.at[pl.ds(off, GCH)], idx_v)
        pltpu.async_copy(table_hbm.at[idx_v], rows_v, sem).wait()
        pltpu.sync_copy(rows_v, out_hbm.at[pl.ds(off, GCH)])


@functools.partial(
    pl.kernel,
    out_type=jax.ShapeDtypeStruct((2, NUM_ET, NPAD, D), jnp.float32),
    mesh=_mesh,
    scratch_types=(
        [pltpu.VMEM_SHARED((NPAD, D), jnp.float32)]
        + [pltpu.VMEM((NCH, CH), jnp.int32)] * 2
        + [pltpu.VMEM((CH,), jnp.int32)] * (2 * NB)
        + [pltpu.VMEM((CH, D), jnp.float32)] * NB
        + [pltpu.SemaphoreType.DMA] * (2 * NB)
    ),
)
def _seg_sums(h_hbm, src0, dst0, src1, dst1, src2, dst2, zeros_hbm, out_hbm,
              acc_sh, src_all, dst_all, *bufs):
    srcb = list(bufs[0:NB])
    dstb = list(bufs[NB:2 * NB])
    rows = list(bufs[2 * NB:3 * NB])
    gsem = list(bufs[3 * NB:4 * NB])
    ssem = list(bufs[4 * NB:5 * NB])

    c = lax.axis_index("c")
    s = lax.axis_index("s")
    wid = s * 2 + c
    tile_lo = s * ROWS_PER_TILE

    def copy_idx(dst_buf, src_ref, k):
        for j in range(CH // 16):
            dst_buf[pl.ds(j * 16, 16)] = src_ref[k, pl.ds(j * 16, 16)]

    def wait_rows(sem, buf):
        pltpu.make_async_copy(h_hbm.at[pl.ds(0, CH)], buf, sem).wait()

    edge_lists = ((src0, dst0), (src1, dst1), (src2, dst2))
    for t in range(NUM_ET):
        src_hbm, dst_hbm = edge_lists[t]
        pltpu.sync_copy(src_hbm.at[wid], src_all)
        pltpu.sync_copy(dst_hbm.at[wid], dst_all)
        pltpu.sync_copy(zeros_hbm, acc_sh.at[pl.ds(tile_lo, ROWS_PER_TILE)])
        plsc.subcore_barrier()

        pending = [False] * NB

        def fire_gather(k):
            b = k % NB
            if pending[b]:
                wait_rows(ssem[b], rows[b])
                pending[b] = False
            copy_idx(srcb[b], src_all, k)
            copy_idx(dstb[b], dst_all, k)
            pltpu.async_copy(h_hbm.at[srcb[b]], rows[b], gsem[b])

        def fire_scatter(k):
            b = k % NB
            wait_rows(gsem[b], rows[b])
            pltpu.async_copy(rows[b], acc_sh.at[dstb[b]], ssem[b], add=True)
            pending[b] = True

        fire_gather(0)
        fire_gather(1)
        for k in range(2, NCH):
            fire_scatter(k - 2)
            fire_gather(k)
        fire_scatter(NCH - 2)
        fire_scatter(NCH - 1)
        for b in range(NB):
            if pending[b]:
                wait_rows(ssem[b], rows[b])
                pending[b] = False
        plsc.subcore_barrier()
        pltpu.sync_copy(
            acc_sh.at[pl.ds(tile_lo, ROWS_PER_TILE)],
            out_hbm.at[c, t, pl.ds(tile_lo, ROWS_PER_TILE)],
        )


BM = 512


def _layer_body(h_ref, s_ref, wself_ref, wmsg_ref, out_ref):
    acc = jnp.dot(h_ref[...], wself_ref[...], preferred_element_type=jnp.float32)
    for t in range(NUM_ET):
        st = s_ref[0, t] + s_ref[1, t]
        acc += jnp.dot(st, wmsg_ref[t], preferred_element_type=jnp.float32)
    out_ref[...] = jnp.maximum(acc, 0.0)


def _tc_layer(h, S, wself, wmsg):
    return pl.pallas_call(
        _layer_body,
        grid=(NPAD // BM,),
        in_specs=[
            pl.BlockSpec((BM, D), lambda i: (i, 0)),
            pl.BlockSpec((2, NUM_ET, BM, D), lambda i: (0, 0, i, 0)),
            pl.BlockSpec((D, D), lambda i: (0, 0)),
            pl.BlockSpec((NUM_ET, D, D), lambda i: (0, 0, 0)),
        ],
        out_specs=pl.BlockSpec((BM, D), lambda i: (i, 0)),
        out_shape=jax.ShapeDtypeStruct((NPAD, D), jnp.float32),
    )(h, S, wself, wmsg)


def kernel(node_label_ids, adjacency_list_0, adjacency_list_1, adjacency_list_2,
           node_to_graph_map, num_graphs, emb_table, W_msg, W_self):
    ids = jnp.zeros((NPAD,), jnp.int32).at[:N_NODES].set(
        node_label_ids.astype(jnp.int32))
    srcs, dsts = [], []
    for a in (adjacency_list_0, adjacency_list_1, adjacency_list_2):
        a = a.astype(jnp.int32)
        srcs.append(jnp.zeros((EPAD,), jnp.int32).at[:E_PER].set(a[:, 0])
                    .reshape(NW, NCH, CH))
        dsts.append(jnp.full((EPAD,), NPAD - 1, jnp.int32).at[:E_PER].set(a[:, 1])
                    .reshape(NW, NCH, CH))
    zeros = jnp.zeros((ROWS_PER_TILE, D), jnp.float32)

    h = _emb_gather(emb_table, ids)
    for layer in range(L):
        S = _seg_sums(h, srcs[0], dsts[0], srcs[1], dsts[1], srcs[2], dsts[2],
                      zeros)
        h = _tc_layer(h, S, W_self[layer], W_msg[layer])
    return h[:N_NODES]


# speedup vs baseline: 8.0029x; 5.6371x over previous
import functools

import jax
import jax.numpy as jnp
from jax import lax
from jax.experimental import pallas as pl
from jax.experimental.pallas import tpu as pltpu
from jax.experimental.pallas import tpu_sc as plsc

N_NODES = 10000
D = 128
NUM_ET = 3
E_PER = 106667
L = 2

NW = 32
NPAD = 10240
ROWS_PER_W = NPAD // NW
ROWS_PER_TILE = NPAD // 16

CH = 128
NCH = 28
EW = NCH * CH
EPAD = NW * EW
NB = 2

GCH = 80
NGCH = ROWS_PER_W // GCH

_mesh = plsc.VectorSubcoreMesh(core_axis_name="c", subcore_axis_name="s")


@functools.partial(
    pl.kernel,
    out_type=jax.ShapeDtypeStruct((NPAD, D), jnp.float32),
    mesh=_mesh,
    scratch_types=[
        pltpu.VMEM((GCH,), jnp.int32),
        pltpu.VMEM((GCH, D), jnp.float32),
        pltpu.SemaphoreType.DMA,
    ],
)
def _emb_gather(table_hbm, ids_hbm, out_hbm, idx_v, rows_v, sem):
    c = lax.axis_index("c")
    s = lax.axis_index("s")
    wid = s * 2 + c
    base = wid * ROWS_PER_W
    for k in range(NGCH):
        off = base + k * GCH
        pltpu.sync_copy(ids_hbm.at[pl.ds(off, GCH)], idx_v)
        pltpu.async_copy(table_hbm.at[idx_v], rows_v, sem).wait()
        pltpu.sync_copy(rows_v, out_hbm.at[pl.ds(off, GCH)])


@functools.partial(
    pl.kernel,
    out_type=jax.ShapeDtypeStruct((2, NUM_ET, NPAD, D), jnp.float32),
    mesh=_mesh,
    scratch_types=(
        [pltpu.VMEM_SHARED((NPAD, D), jnp.float32)]
        + [pltpu.VMEM((NCH, CH), jnp.int32)] * 2
        + [pltpu.VMEM((CH,), jnp.int32)] * (2 * NB)
        + [pltpu.VMEM((CH, D), jnp.float32)] * NB
        + [pltpu.SemaphoreType.DMA] * (2 * NB)
    ),
)
def _seg_sums(h_hbm, src0, dst0, src1, dst1, src2, dst2, zeros_hbm, out_hbm,
              acc_sh, src_all, dst_all, *bufs):
    srcb = list(bufs[0:NB])
    dstb = list(bufs[NB:2 * NB])
    rows = list(bufs[2 * NB:3 * NB])
    gsem = list(bufs[3 * NB:4 * NB])
    ssem = list(bufs[4 * NB:5 * NB])

    c = lax.axis_index("c")
    s = lax.axis_index("s")
    wid = s * 2 + c
    tile_lo = s * ROWS_PER_TILE

    def copy_idx(dst_buf, src_ref, k):
        for j in range(CH // 16):
            dst_buf[pl.ds(j * 16, 16)] = src_ref[k, pl.ds(j * 16, 16)]

    def wait_rows(sem, buf):
        pltpu.make_async_copy(h_hbm.at[pl.ds(0, CH)], buf, sem).wait()

    edge_lists = ((src0, dst0), (src1, dst1), (src2, dst2))
    for t in range(NUM_ET):
        src_hbm, dst_hbm = edge_lists[t]
        pltpu.sync_copy(src_hbm.at[wid], src_all)
        pltpu.sync_copy(dst_hbm.at[wid], dst_all)
        pltpu.sync_copy(zeros_hbm, acc_sh.at[pl.ds(tile_lo, ROWS_PER_TILE)])
        plsc.subcore_barrier()

        pending = [False] * NB

        def fire_gather(k):
            b = k % NB
            if pending[b]:
                wait_rows(ssem[b], rows[b])
                pending[b] = False
            copy_idx(srcb[b], src_all, k)
            copy_idx(dstb[b], dst_all, k)
            pltpu.async_copy(h_hbm.at[srcb[b]], rows[b], gsem[b])

        def fire_scatter(k):
            b = k % NB
            wait_rows(gsem[b], rows[b])
            pltpu.async_copy(rows[b], acc_sh.at[dstb[b]], ssem[b], add=True)
            pending[b] = True

        fire_gather(0)
        fire_gather(1)
        for k in range(2, NCH):
            fire_scatter(k - 2)
            fire_gather(k)
        fire_scatter(NCH - 2)
        fire_scatter(NCH - 1)
        for b in range(NB):
            if pending[b]:
                wait_rows(ssem[b], rows[b])
                pending[b] = False
        plsc.subcore_barrier()
        pltpu.sync_copy(
            acc_sh.at[pl.ds(tile_lo, ROWS_PER_TILE)],
            out_hbm.at[c, t, pl.ds(tile_lo, ROWS_PER_TILE)],
        )


BM = 512


def _layer_body(h_ref, s_ref, wself_ref, wmsg_ref, out_ref):
    acc = jnp.dot(h_ref[...], wself_ref[...], preferred_element_type=jnp.float32)
    for t in range(NUM_ET):
        st = s_ref[0, t] + s_ref[1, t]
        acc += jnp.dot(st, wmsg_ref[t], preferred_element_type=jnp.float32)
    out_ref[...] = jnp.maximum(acc, 0.0)


def _tc_layer(h, S, wself, wmsg):
    return pl.pallas_call(
        _layer_body,
        grid=(NPAD // BM,),
        in_specs=[
            pl.BlockSpec((BM, D), lambda i: (i, 0)),
            pl.BlockSpec((2, NUM_ET, BM, D), lambda i: (0, 0, i, 0)),
            pl.BlockSpec((D, D), lambda i: (0, 0)),
            pl.BlockSpec((NUM_ET, D, D), lambda i: (0, 0, 0)),
        ],
        out_specs=pl.BlockSpec((BM, D), lambda i: (i, 0)),
        out_shape=jax.ShapeDtypeStruct((NPAD, D), jnp.float32),
    )(h, S, wself, wmsg)


def kernel(node_label_ids, adjacency_list_0, adjacency_list_1, adjacency_list_2,
           node_to_graph_map, num_graphs, emb_table, W_msg, W_self):
    ids = jnp.zeros((NPAD,), jnp.int32).at[:N_NODES].set(
        node_label_ids.astype(jnp.int32))
    srcs, dsts = [], []
    iota = jnp.arange(EPAD, dtype=jnp.int32)
    pad_src = iota % N_NODES
    pad_dst = N_NODES + iota % (NPAD - N_NODES)
    for a in (adjacency_list_0, adjacency_list_1, adjacency_list_2):
        a = a.astype(jnp.int32)
        srcs.append(pad_src.at[:E_PER].set(a[:, 0]).reshape(NW, NCH, CH))
        dsts.append(pad_dst.at[:E_PER].set(a[:, 1]).reshape(NW, NCH, CH))
    zeros = jnp.zeros((ROWS_PER_TILE, D), jnp.float32)

    h = _emb_gather(emb_table, ids)
    for layer in range(L):
        S = _seg_sums(h, srcs[0], dsts[0], srcs[1], dsts[1], srcs[2], dsts[2],
                      zeros)
        h = _tc_layer(h, S, W_self[layer], W_msg[layer])
    return h[:N_NODES]


# speedup vs baseline: 8.1437x; 1.0176x over previous
import functools

import jax
import jax.numpy as jnp
from jax import lax
from jax.experimental import pallas as pl
from jax.experimental.pallas import tpu as pltpu
from jax.experimental.pallas import tpu_sc as plsc

N_NODES = 10000
D = 128
NUM_ET = 3
E_PER = 106667
L = 2

NW = 32
NPAD = 10240
ROWS_PER_W = NPAD // NW
ROWS_PER_TILE = NPAD // 16

CH = 128
HC = CH // 2
NCH = 27
EW = NCH * CH
EPAD = NW * EW
NB = 2

GCH = 80
NGCH = ROWS_PER_W // GCH

_mesh = plsc.VectorSubcoreMesh(core_axis_name="c", subcore_axis_name="s")


@functools.partial(
    pl.kernel,
    out_type=jax.ShapeDtypeStruct((NPAD, D), jnp.float32),
    mesh=_mesh,
    scratch_types=[
        pltpu.VMEM((GCH,), jnp.int32),
        pltpu.VMEM((GCH, D), jnp.float32),
        pltpu.SemaphoreType.DMA,
    ],
)
def _emb_gather(table_hbm, ids_hbm, out_hbm, idx_v, rows_v, sem):
    c = lax.axis_index("c")
    s = lax.axis_index("s")
    wid = s * 2 + c
    base = wid * ROWS_PER_W
    for k in range(NGCH):
        off = base + k * GCH
        pltpu.sync_copy(ids_hbm.at[pl.ds(off, GCH)], idx_v)
        pltpu.async_copy(table_hbm.at[idx_v], rows_v, sem).wait()
        pltpu.sync_copy(rows_v, out_hbm.at[pl.ds(off, GCH)])


@functools.partial(
    pl.kernel,
    out_type=jax.ShapeDtypeStruct((2, NUM_ET, NPAD, D), jnp.float32),
    mesh=_mesh,
    scratch_types=(
        [pltpu.VMEM_SHARED((NPAD, D), jnp.float32)]
        + [pltpu.VMEM((NCH, CH), jnp.int32)] * 2
        + [pltpu.VMEM((HC,), jnp.int32)] * (4 * NB)
        + [pltpu.VMEM((CH, D), jnp.float32)] * NB
        + [pltpu.SemaphoreType.DMA] * (2 * NB)
    ),
)
def _seg_sums(h_hbm, src0, dst0, src1, dst1, src2, dst2, zeros_hbm, out_hbm,
              acc_sh, src_all, dst_all, *bufs):
    srcb = [(bufs[2 * b], bufs[2 * b + 1]) for b in range(NB)]
    dstb = [(bufs[2 * NB + 2 * b], bufs[2 * NB + 2 * b + 1]) for b in range(NB)]
    rows = list(bufs[4 * NB:5 * NB])
    gsem = list(bufs[5 * NB:6 * NB])
    ssem = list(bufs[6 * NB:7 * NB])

    c = lax.axis_index("c")
    s = lax.axis_index("s")
    wid = s * 2 + c
    tile_lo = s * ROWS_PER_TILE

    def copy_idx(half_bufs, src_ref, k):
        d0, d1 = half_bufs
        for j in range(HC // 16):
            d0[pl.ds(j * 16, 16)] = src_ref[k, pl.ds(j * 16, 16)]
            d1[pl.ds(j * 16, 16)] = src_ref[k, pl.ds(HC + j * 16, 16)]

    def wait_rows(sem, buf):
        pltpu.make_async_copy(h_hbm.at[pl.ds(0, CH)], buf, sem).wait()

    edge_lists = ((src0, dst0), (src1, dst1), (src2, dst2))
    for t in range(NUM_ET):
        src_hbm, dst_hbm = edge_lists[t]
        pltpu.sync_copy(src_hbm.at[wid], src_all)
        pltpu.sync_copy(dst_hbm.at[wid], dst_all)
        pltpu.sync_copy(zeros_hbm, acc_sh.at[pl.ds(tile_lo, ROWS_PER_TILE)])
        plsc.subcore_barrier()

        pending = [False] * NB

        def fire_gather(k):
            b = k % NB
            if pending[b]:
                wait_rows(ssem[b], rows[b])
                pending[b] = False
            copy_idx(srcb[b], src_all, k)
            copy_idx(dstb[b], dst_all, k)
            pltpu.async_copy(h_hbm.at[srcb[b][0]], rows[b].at[pl.ds(0, HC)],
                             gsem[b])
            pltpu.async_copy(h_hbm.at[srcb[b][1]], rows[b].at[pl.ds(HC, HC)],
                             gsem[b])

        def fire_scatter(k):
            b = k % NB
            wait_rows(gsem[b], rows[b])
            pltpu.async_copy(rows[b].at[pl.ds(0, HC)], acc_sh.at[dstb[b][0]],
                             ssem[b], add=True)
            pltpu.async_copy(rows[b].at[pl.ds(HC, HC)], acc_sh.at[dstb[b][1]],
                             ssem[b], add=True)
            pending[b] = True

        fire_gather(0)
        fire_gather(1)
        for k in range(2, NCH):
            fire_scatter(k - 2)
            fire_gather(k)
        fire_scatter(NCH - 2)
        fire_scatter(NCH - 1)
        for b in range(NB):
            if pending[b]:
                wait_rows(ssem[b], rows[b])
                pending[b] = False
        plsc.subcore_barrier()
        pltpu.sync_copy(
            acc_sh.at[pl.ds(tile_lo, ROWS_PER_TILE)],
            out_hbm.at[c, t, pl.ds(tile_lo, ROWS_PER_TILE)],
        )


BM = 512


def _layer_body(h_ref, s_ref, wself_ref, wmsg_ref, out_ref):
    acc = jnp.dot(h_ref[...], wself_ref[...], preferred_element_type=jnp.float32)
    for t in range(NUM_ET):
        st = s_ref[0, t] + s_ref[1, t]
        acc += jnp.dot(st, wmsg_ref[t], preferred_element_type=jnp.float32)
    out_ref[...] = jnp.maximum(acc, 0.0)


def _tc_layer(h, S, wself, wmsg):
    return pl.pallas_call(
        _layer_body,
        grid=(NPAD // BM,),
        in_specs=[
            pl.BlockSpec((BM, D), lambda i: (i, 0)),
            pl.BlockSpec((2, NUM_ET, BM, D), lambda i: (0, 0, i, 0)),
            pl.BlockSpec((D, D), lambda i: (0, 0)),
            pl.BlockSpec((NUM_ET, D, D), lambda i: (0, 0, 0)),
        ],
        out_specs=pl.BlockSpec((BM, D), lambda i: (i, 0)),
        out_shape=jax.ShapeDtypeStruct((NPAD, D), jnp.float32),
    )(h, S, wself, wmsg)


def kernel(node_label_ids, adjacency_list_0, adjacency_list_1, adjacency_list_2,
           node_to_graph_map, num_graphs, emb_table, W_msg, W_self):
    ids = jnp.zeros((NPAD,), jnp.int32).at[:N_NODES].set(
        node_label_ids.astype(jnp.int32))
    srcs, dsts = [], []
    iota = jnp.arange(EPAD, dtype=jnp.int32)
    pad_src = iota % N_NODES
    pad_dst = N_NODES + iota % (NPAD - N_NODES)
    for a in (adjacency_list_0, adjacency_list_1, adjacency_list_2):
        a = a.astype(jnp.int32)
        srcs.append(pad_src.at[:E_PER].set(a[:, 0]).reshape(NW, NCH, CH))
        dsts.append(pad_dst.at[:E_PER].set(a[:, 1]).reshape(NW, NCH, CH))
    zeros = jnp.zeros((ROWS_PER_TILE, D), jnp.float32)

    h = _emb_gather(emb_table, ids)
    for layer in range(L):
        S = _seg_sums(h, srcs[0], dsts[0], srcs[1], dsts[1], srcs[2], dsts[2],
                      zeros)
        h = _tc_layer(h, S, W_self[layer], W_msg[layer])
    return h[:N_NODES]


# speedup vs baseline: 8.2721x; 1.0158x over previous
import functools

import jax
import jax.numpy as jnp
from jax import lax
from jax.experimental import pallas as pl
from jax.experimental.pallas import tpu as pltpu
from jax.experimental.pallas import tpu_sc as plsc

N_NODES = 10000
D = 128
NUM_ET = 3
E_PER = 106667
L = 2

NW = 32
NPAD = 10240
ROWS_PER_W = NPAD // NW
ROWS_PER_TILE = NPAD // 16

CH = 128
HC = CH // 2
NCH = 27
EW = NCH * CH
EPAD = NW * EW
NB = 2

GCH = 80
NGCH = ROWS_PER_W // GCH

_mesh = plsc.VectorSubcoreMesh(core_axis_name="c", subcore_axis_name="s")


@functools.partial(
    pl.kernel,
    out_type=jax.ShapeDtypeStruct((NPAD, D), jnp.float32),
    mesh=_mesh,
    scratch_types=[
        pltpu.VMEM((GCH,), jnp.int32),
        pltpu.VMEM((GCH, D), jnp.float32),
        pltpu.SemaphoreType.DMA,
    ],
)
def _emb_gather(table_hbm, ids_hbm, out_hbm, idx_v, rows_v, sem):
    c = lax.axis_index("c")
    s = lax.axis_index("s")
    wid = s * 2 + c
    base = wid * ROWS_PER_W
    for k in range(NGCH):
        off = base + k * GCH
        pltpu.sync_copy(ids_hbm.at[pl.ds(off, GCH)], idx_v)
        pltpu.async_copy(table_hbm.at[idx_v], rows_v, sem).wait()
        pltpu.sync_copy(rows_v, out_hbm.at[pl.ds(off, GCH)])


@functools.partial(
    pl.kernel,
    out_type=jax.ShapeDtypeStruct((2, NUM_ET, NPAD, D), jnp.float32),
    mesh=_mesh,
    scratch_types=(
        [pltpu.VMEM_SHARED((NPAD, D), jnp.float32)]
        + [pltpu.VMEM((NCH, CH), jnp.int32)] * 2
        + [pltpu.VMEM((HC,), jnp.int32)] * (4 * NB)
        + [pltpu.VMEM((CH, D), jnp.float32)] * NB
        + [pltpu.SemaphoreType.DMA] * (2 * NB)
    ),
)
def _seg_sums(h_hbm, src0, dst0, src1, dst1, src2, dst2, zeros_hbm, out_hbm,
              acc_sh, src_all, dst_all, *bufs):
    srcb = [(bufs[2 * b], bufs[2 * b + 1]) for b in range(NB)]
    dstb = [(bufs[2 * NB + 2 * b], bufs[2 * NB + 2 * b + 1]) for b in range(NB)]
    rows = list(bufs[4 * NB:5 * NB])
    gsem = list(bufs[5 * NB:6 * NB])
    ssem = list(bufs[6 * NB:7 * NB])

    c = lax.axis_index("c")
    s = lax.axis_index("s")
    wid = s * 2 + c
    tile_lo = s * ROWS_PER_TILE

    def copy_idx(half_bufs, src_ref, k):
        d0, d1 = half_bufs
        for j in range(HC // 16):
            d0[pl.ds(j * 16, 16)] = src_ref[k, pl.ds(j * 16, 16)]
            d1[pl.ds(j * 16, 16)] = src_ref[k, pl.ds(HC + j * 16, 16)]

    def wait_rows(sem, buf):
        pltpu.make_async_copy(h_hbm.at[pl.ds(0, CH)], buf, sem).wait()

    pending = [False] * NB

    def fire_gather(k):
        b = k % NB
        if pending[b]:
            wait_rows(ssem[b], rows[b])
            pending[b] = False
        copy_idx(srcb[b], src_all, k)
        copy_idx(dstb[b], dst_all, k)
        pltpu.async_copy(h_hbm.at[srcb[b][0]], rows[b].at[pl.ds(0, HC)],
                         gsem[b])
        pltpu.async_copy(h_hbm.at[srcb[b][1]], rows[b].at[pl.ds(HC, HC)],
                         gsem[b])

    def fire_scatter(k):
        b = k % NB
        wait_rows(gsem[b], rows[b])
        pltpu.async_copy(rows[b].at[pl.ds(0, HC)], acc_sh.at[dstb[b][0]],
                         ssem[b], add=True)
        pltpu.async_copy(rows[b].at[pl.ds(HC, HC)], acc_sh.at[dstb[b][1]],
                         ssem[b], add=True)
        pending[b] = True

    def stage(t):
        src_hbm, dst_hbm = edge_lists[t]
        pltpu.sync_copy(src_hbm.at[wid], src_all)
        pltpu.sync_copy(dst_hbm.at[wid], dst_all)

    def zero_slice():
        pltpu.sync_copy(zeros_hbm, acc_sh.at[pl.ds(tile_lo, ROWS_PER_TILE)])

    def flush(t):
        pltpu.sync_copy(
            acc_sh.at[pl.ds(tile_lo, ROWS_PER_TILE)],
            out_hbm.at[c, t, pl.ds(tile_lo, ROWS_PER_TILE)],
        )

    edge_lists = ((src0, dst0), (src1, dst1), (src2, dst2))
    stage(0)
    fire_gather(0)
    fire_gather(1)
    zero_slice()
    plsc.subcore_barrier()
    for t in range(NUM_ET):
        for k in range(2, NCH):
            fire_scatter(k - 2)
            fire_gather(k)
        fire_scatter(NCH - 2)
        fire_scatter(NCH - 1)
        for b in range(NB):
            if pending[b]:
                wait_rows(ssem[b], rows[b])
                pending[b] = False
        if t + 1 < NUM_ET:
            stage(t + 1)
            fire_gather(0)
            fire_gather(1)
        plsc.subcore_barrier()
        flush(t)
        if t + 1 < NUM_ET:
            zero_slice()
            plsc.subcore_barrier()


BM = 512


def _layer_body(h_ref, s_ref, wself_ref, wmsg_ref, out_ref):
    acc = jnp.dot(h_ref[...], wself_ref[...], preferred_element_type=jnp.float32)
    for t in range(NUM_ET):
        st = s_ref[0, t] + s_ref[1, t]
        acc += jnp.dot(st, wmsg_ref[t], preferred_element_type=jnp.float32)
    out_ref[...] = jnp.maximum(acc, 0.0)


def _tc_layer(h, S, wself, wmsg):
    return pl.pallas_call(
        _layer_body,
        grid=(NPAD // BM,),
        in_specs=[
            pl.BlockSpec((BM, D), lambda i: (i, 0)),
            pl.BlockSpec((2, NUM_ET, BM, D), lambda i: (0, 0, i, 0)),
            pl.BlockSpec((D, D), lambda i: (0, 0)),
            pl.BlockSpec((NUM_ET, D, D), lambda i: (0, 0, 0)),
        ],
        out_specs=pl.BlockSpec((BM, D), lambda i: (i, 0)),
        out_shape=jax.ShapeDtypeStruct((NPAD, D), jnp.float32),
    )(h, S, wself, wmsg)


def kernel(node_label_ids, adjacency_list_0, adjacency_list_1, adjacency_list_2,
           node_to_graph_map, num_graphs, emb_table, W_msg, W_self):
    ids = jnp.zeros((NPAD,), jnp.int32).at[:N_NODES].set(
        node_label_ids.astype(jnp.int32))
    srcs, dsts = [], []
    iota = jnp.arange(EPAD, dtype=jnp.int32)
    pad_src = iota % N_NODES
    pad_dst = N_NODES + iota % (NPAD - N_NODES)
    for a in (adjacency_list_0, adjacency_list_1, adjacency_list_2):
        a = a.astype(jnp.int32)
        srcs.append(pad_src.at[:E_PER].set(a[:, 0]).reshape(NW, NCH, CH))
        dsts.append(pad_dst.at[:E_PER].set(a[:, 1]).reshape(NW, NCH, CH))
    zeros = jnp.zeros((ROWS_PER_TILE, D), jnp.float32)

    h = _emb_gather(emb_table, ids)
    for layer in range(L):
        S = _seg_sums(h, srcs[0], dsts[0], srcs[1], dsts[1], srcs[2], dsts[2],
                      zeros)
        h = _tc_layer(h, S, W_self[layer], W_msg[layer])
    return h[:N_NODES]


# speedup vs baseline: 8.3194x; 1.0057x over previous
import functools

import jax
import jax.numpy as jnp
from jax import lax
from jax.experimental import pallas as pl
from jax.experimental.pallas import tpu as pltpu
from jax.experimental.pallas import tpu_sc as plsc

N_NODES = 10000
D = 128
NUM_ET = 3
E_PER = 106667
L = 2

NW = 32
NPAD = 10240
ROWS_PER_W = NPAD // NW
ROWS_PER_TILE = NPAD // 16

CH = 128
HC = CH // 2
NCH = 27
EW = NCH * CH
EPAD = NW * EW
NB = 2

GCH = 80
NGCH = ROWS_PER_W // GCH

_mesh = plsc.VectorSubcoreMesh(core_axis_name="c", subcore_axis_name="s")


@functools.partial(
    pl.kernel,
    out_type=jax.ShapeDtypeStruct((NPAD, D), jnp.float32),
    mesh=_mesh,
    scratch_types=[
        pltpu.VMEM((GCH,), jnp.int32),
        pltpu.VMEM((GCH,), jnp.int32),
        pltpu.VMEM((GCH, D), jnp.float32),
        pltpu.VMEM((GCH, D), jnp.float32),
        pltpu.SemaphoreType.DMA,
        pltpu.SemaphoreType.DMA,
    ],
)
def _emb_gather(table_hbm, ids_hbm, out_hbm, idxA, idxB, rowsA, rowsB,
                semA, semB):
    c = lax.axis_index("c")
    s = lax.axis_index("s")
    wid = s * 2 + c
    base = wid * ROWS_PER_W
    idx = (idxA, idxB)
    rows = (rowsA, rowsB)
    sem = (semA, semB)
    for k in range(NGCH):
        b = k % 2
        off = base + k * GCH
        pltpu.sync_copy(ids_hbm.at[pl.ds(off, GCH)], idx[b])
        pltpu.async_copy(table_hbm.at[idx[b]], rows[b], sem[b])
        if k > 0:
            pb = (k - 1) % 2
            poff = base + (k - 1) * GCH
            pltpu.make_async_copy(
                table_hbm.at[pl.ds(0, GCH)], rows[pb], sem[pb]).wait()
            pltpu.sync_copy(rows[pb], out_hbm.at[pl.ds(poff, GCH)])
    lb = (NGCH - 1) % 2
    pltpu.make_async_copy(
        table_hbm.at[pl.ds(0, GCH)], rows[lb], sem[lb]).wait()
    pltpu.sync_copy(rows[lb], out_hbm.at[pl.ds(base + (NGCH - 1) * GCH, GCH)])


@functools.partial(
    pl.kernel,
    out_type=jax.ShapeDtypeStruct((2, NUM_ET, NPAD, D), jnp.float32),
    mesh=_mesh,
    scratch_types=(
        [pltpu.VMEM_SHARED((NPAD, D), jnp.float32)]
        + [pltpu.VMEM((NCH, CH), jnp.int32)] * 2
        + [pltpu.VMEM((HC,), jnp.int32)] * (4 * NB)
        + [pltpu.VMEM((CH, D), jnp.float32)] * NB
        + [pltpu.SemaphoreType.DMA] * (2 * NB)
    ),
)
def _seg_sums(h_hbm, src0, dst0, src1, dst1, src2, dst2, zeros_hbm, out_hbm,
              acc_sh, src_all, dst_all, *bufs):
    srcb = [(bufs[2 * b], bufs[2 * b + 1]) for b in range(NB)]
    dstb = [(bufs[2 * NB + 2 * b], bufs[2 * NB + 2 * b + 1]) for b in range(NB)]
    rows = list(bufs[4 * NB:5 * NB])
    gsem = list(bufs[5 * NB:6 * NB])
    ssem = list(bufs[6 * NB:7 * NB])

    c = lax.axis_index("c")
    s = lax.axis_index("s")
    wid = s * 2 + c
    tile_lo = s * ROWS_PER_TILE

    def copy_idx(half_bufs, src_ref, k):
        d0, d1 = half_bufs
        for j in range(HC // 16):
            d0[pl.ds(j * 16, 16)] = src_ref[k, pl.ds(j * 16, 16)]
            d1[pl.ds(j * 16, 16)] = src_ref[k, pl.ds(HC + j * 16, 16)]

    def wait_rows(sem, buf):
        pltpu.make_async_copy(h_hbm.at[pl.ds(0, CH)], buf, sem).wait()

    pending = [False] * NB

    def fire_gather(k):
        b = k % NB
        if pending[b]:
            wait_rows(ssem[b], rows[b])
            pending[b] = False
        copy_idx(srcb[b], src_all, k)
        copy_idx(dstb[b], dst_all, k)
        pltpu.async_copy(h_hbm.at[srcb[b][0]], rows[b].at[pl.ds(0, HC)],
                         gsem[b])
        pltpu.async_copy(h_hbm.at[srcb[b][1]], rows[b].at[pl.ds(HC, HC)],
                         gsem[b])

    def fire_scatter(k):
        b = k % NB
        wait_rows(gsem[b], rows[b])
        pltpu.async_copy(rows[b].at[pl.ds(0, HC)], acc_sh.at[dstb[b][0]],
                         ssem[b], add=True)
        pltpu.async_copy(rows[b].at[pl.ds(HC, HC)], acc_sh.at[dstb[b][1]],
                         ssem[b], add=True)
        pending[b] = True

    def stage(t):
        src_hbm, dst_hbm = edge_lists[t]
        pltpu.sync_copy(src_hbm.at[wid], src_all)
        pltpu.sync_copy(dst_hbm.at[wid], dst_all)

    def zero_slice():
        pltpu.sync_copy(zeros_hbm, acc_sh.at[pl.ds(tile_lo, ROWS_PER_TILE)])

    def flush(t):
        pltpu.sync_copy(
            acc_sh.at[pl.ds(tile_lo, ROWS_PER_TILE)],
            out_hbm.at[c, t, pl.ds(tile_lo, ROWS_PER_TILE)],
        )

    edge_lists = ((src0, dst0), (src1, dst1), (src2, dst2))
    stage(0)
    fire_gather(0)
    fire_gather(1)
    zero_slice()
    plsc.subcore_barrier()
    for t in range(NUM_ET):
        for k in range(2, NCH):
            fire_scatter(k - 2)
            fire_gather(k)
        fire_scatter(NCH - 2)
        fire_scatter(NCH - 1)
        for b in range(NB):
            if pending[b]:
                wait_rows(ssem[b], rows[b])
                pending[b] = False
        if t + 1 < NUM_ET:
            stage(t + 1)
            fire_gather(0)
            fire_gather(1)
        plsc.subcore_barrier()
        flush(t)
        if t + 1 < NUM_ET:
            zero_slice()
            plsc.subcore_barrier()


BM = 512


def _layer_body(h_ref, s_ref, wself_ref, wmsg_ref, out_ref):
    acc = jnp.dot(h_ref[...], wself_ref[...], preferred_element_type=jnp.float32)
    for t in range(NUM_ET):
        st = s_ref[0, t] + s_ref[1, t]
        acc += jnp.dot(st, wmsg_ref[t], preferred_element_type=jnp.float32)
    out_ref[...] = jnp.maximum(acc, 0.0)


def _tc_layer(h, S, wself, wmsg):
    return pl.pallas_call(
        _layer_body,
        grid=(NPAD // BM,),
        in_specs=[
            pl.BlockSpec((BM, D), lambda i: (i, 0)),
            pl.BlockSpec((2, NUM_ET, BM, D), lambda i: (0, 0, i, 0)),
            pl.BlockSpec((D, D), lambda i: (0, 0)),
            pl.BlockSpec((NUM_ET, D, D), lambda i: (0, 0, 0)),
        ],
        out_specs=pl.BlockSpec((BM, D), lambda i: (i, 0)),
        out_shape=jax.ShapeDtypeStruct((NPAD, D), jnp.float32),
    )(h, S, wself, wmsg)


def kernel(node_label_ids, adjacency_list_0, adjacency_list_1, adjacency_list_2,
           node_to_graph_map, num_graphs, emb_table, W_msg, W_self):
    ids = jnp.zeros((NPAD,), jnp.int32).at[:N_NODES].set(
        node_label_ids.astype(jnp.int32))
    srcs, dsts = [], []
    iota = jnp.arange(EPAD, dtype=jnp.int32)
    pad_src = iota % N_NODES
    pad_dst = N_NODES + iota % (NPAD - N_NODES)
    for a in (adjacency_list_0, adjacency_list_1, adjacency_list_2):
        a = a.astype(jnp.int32)
        srcs.append(pad_src.at[:E_PER].set(a[:, 0]).reshape(NW, NCH, CH))
        dsts.append(pad_dst.at[:E_PER].set(a[:, 1]).reshape(NW, NCH, CH))
    zeros = jnp.zeros((ROWS_PER_TILE, D), jnp.float32)

    h = _emb_gather(emb_table, ids)
    for layer in range(L):
        S = _seg_sums(h, srcs[0], dsts[0], srcs[1], dsts[1], srcs[2], dsts[2],
                      zeros)
        h = _tc_layer(h, S, W_self[layer], W_msg[layer])
    return h[:N_NODES]


# speedup vs baseline: 8.5904x; 1.0326x over previous
import functools

import jax
import jax.numpy as jnp
from jax import lax
from jax.experimental import pallas as pl
from jax.experimental.pallas import tpu as pltpu
from jax.experimental.pallas import tpu_sc as plsc

N_NODES = 10000
D = 128
NUM_ET = 3
E_PER = 106667
L = 2

NW = 32
NPAD = 10240
ROWS_PER_W = NPAD // NW
ROWS_PER_TILE = NPAD // 16

CH = 128
HC = CH // 2
NCH = 27
EW = NCH * CH
EPAD = NW * EW
NB = 2

GCH = 80
NGCH = ROWS_PER_W // GCH

_mesh = plsc.VectorSubcoreMesh(core_axis_name="c", subcore_axis_name="s")


@functools.partial(
    pl.kernel,
    out_type=jax.ShapeDtypeStruct((NPAD, D), jnp.float32),
    mesh=_mesh,
    scratch_types=[
        pltpu.VMEM((GCH,), jnp.int32),
        pltpu.VMEM((GCH,), jnp.int32),
        pltpu.VMEM((GCH, D), jnp.float32),
        pltpu.VMEM((GCH, D), jnp.float32),
        pltpu.SemaphoreType.DMA,
        pltpu.SemaphoreType.DMA,
    ],
)
def _emb_gather(table_hbm, ids_hbm, out_hbm, idxA, idxB, rowsA, rowsB,
                semA, semB):
    c = lax.axis_index("c")
    s = lax.axis_index("s")
    wid = s * 2 + c
    base = wid * ROWS_PER_W
    idx = (idxA, idxB)
    rows = (rowsA, rowsB)
    sem = (semA, semB)
    for k in range(NGCH):
        b = k % 2
        off = base + k * GCH
        pltpu.sync_copy(ids_hbm.at[pl.ds(off, GCH)], idx[b])
        pltpu.async_copy(table_hbm.at[idx[b]], rows[b], sem[b])
        if k > 0:
            pb = (k - 1) % 2
            poff = base + (k - 1) * GCH
            pltpu.make_async_copy(
                table_hbm.at[pl.ds(0, GCH)], rows[pb], sem[pb]).wait()
            pltpu.sync_copy(rows[pb], out_hbm.at[pl.ds(poff, GCH)])
    lb = (NGCH - 1) % 2
    pltpu.make_async_copy(
        table_hbm.at[pl.ds(0, GCH)], rows[lb], sem[lb]).wait()
    pltpu.sync_copy(rows[lb], out_hbm.at[pl.ds(base + (NGCH - 1) * GCH, GCH)])


@functools.partial(
    pl.kernel,
    out_type=jax.ShapeDtypeStruct((2, NUM_ET, NPAD, D), jnp.float32),
    mesh=_mesh,
    scratch_types=(
        [pltpu.VMEM_SHARED((NPAD, D), jnp.float32)]
        + [pltpu.VMEM((NCH, CH), jnp.int32)] * 2
        + [pltpu.VMEM((HC,), jnp.int32)] * (4 * NB)
        + [pltpu.VMEM((CH, D), jnp.float32)] * NB
        + [pltpu.SemaphoreType.DMA] * (2 * NB)
    ),
)
def _seg_sums(h_hbm, src0, dst0, src1, dst1, src2, dst2, zeros_hbm, out_hbm,
              acc_sh, src_all, dst_all, *bufs):
    srcb = [(bufs[2 * b], bufs[2 * b + 1]) for b in range(NB)]
    dstb = [(bufs[2 * NB + 2 * b], bufs[2 * NB + 2 * b + 1]) for b in range(NB)]
    rows = list(bufs[4 * NB:5 * NB])
    gsem = list(bufs[5 * NB:6 * NB])
    ssem = list(bufs[6 * NB:7 * NB])

    c = lax.axis_index("c")
    s = lax.axis_index("s")
    wid = s * 2 + c
    tile_lo = s * ROWS_PER_TILE

    def copy_idx(half_bufs, src_ref, k):
        d0, d1 = half_bufs
        for j in range(HC // 16):
            d0[pl.ds(j * 16, 16)] = src_ref[k, pl.ds(j * 16, 16)]
            d1[pl.ds(j * 16, 16)] = src_ref[k, pl.ds(HC + j * 16, 16)]

    def wait_rows(sem, buf):
        pltpu.make_async_copy(h_hbm.at[pl.ds(0, CH)], buf, sem).wait()

    pending = [False] * NB

    def fire_gather(k):
        b = k % NB
        if pending[b]:
            wait_rows(ssem[b], rows[b])
            pending[b] = False
        copy_idx(srcb[b], src_all, k)
        copy_idx(dstb[b], dst_all, k)
        pltpu.async_copy(h_hbm.at[srcb[b][0]], rows[b].at[pl.ds(0, HC)],
                         gsem[b])
        pltpu.async_copy(h_hbm.at[srcb[b][1]], rows[b].at[pl.ds(HC, HC)],
                         gsem[b])

    def fire_scatter(k):
        b = k % NB
        wait_rows(gsem[b], rows[b])
        pltpu.async_copy(rows[b].at[pl.ds(0, HC)], acc_sh.at[dstb[b][0]],
                         ssem[b], add=True)
        pltpu.async_copy(rows[b].at[pl.ds(HC, HC)], acc_sh.at[dstb[b][1]],
                         ssem[b], add=True)
        pending[b] = True

    def stage(t):
        src_hbm, dst_hbm = edge_lists[t]
        pltpu.sync_copy(src_hbm.at[wid], src_all)
        pltpu.sync_copy(dst_hbm.at[wid], dst_all)

    def zero_slice():
        pltpu.sync_copy(zeros_hbm, acc_sh.at[pl.ds(tile_lo, ROWS_PER_TILE)])

    def flush(t):
        pltpu.sync_copy(
            acc_sh.at[pl.ds(tile_lo, ROWS_PER_TILE)],
            out_hbm.at[c, t, pl.ds(tile_lo, ROWS_PER_TILE)],
        )

    edge_lists = ((src0, dst0), (src1, dst1), (src2, dst2))
    stage(0)
    fire_gather(0)
    fire_gather(1)
    zero_slice()
    plsc.subcore_barrier()
    for t in range(NUM_ET):
        for k in range(2, NCH):
            fire_scatter(k - 2)
            fire_gather(k)
        fire_scatter(NCH - 2)
        fire_scatter(NCH - 1)
        for b in range(NB):
            if pending[b]:
                wait_rows(ssem[b], rows[b])
                pending[b] = False
        if t + 1 < NUM_ET:
            stage(t + 1)
            fire_gather(0)
            fire_gather(1)
        plsc.subcore_barrier()
        flush(t)
        if t + 1 < NUM_ET:
            zero_slice()
            plsc.subcore_barrier()


BM = 1024


def _layer_body(h_ref, s_ref, wself_ref, wmsg_ref, out_ref):
    acc = jnp.dot(h_ref[...], wself_ref[...], preferred_element_type=jnp.float32)
    for t in range(NUM_ET):
        st = s_ref[0, t] + s_ref[1, t]
        acc += jnp.dot(st, wmsg_ref[t], preferred_element_type=jnp.float32)
    out_ref[...] = jnp.maximum(acc, 0.0)


def _tc_layer(h, S, wself, wmsg):
    return pl.pallas_call(
        _layer_body,
        grid=(NPAD // BM,),
        in_specs=[
            pl.BlockSpec((BM, D), lambda i: (i, 0)),
            pl.BlockSpec((2, NUM_ET, BM, D), lambda i: (0, 0, i, 0)),
            pl.BlockSpec((D, D), lambda i: (0, 0)),
            pl.BlockSpec((NUM_ET, D, D), lambda i: (0, 0, 0)),
        ],
        out_specs=pl.BlockSpec((BM, D), lambda i: (i, 0)),
        out_shape=jax.ShapeDtypeStruct((NPAD, D), jnp.float32),
    )(h, S, wself, wmsg)


def kernel(node_label_ids, adjacency_list_0, adjacency_list_1, adjacency_list_2,
           node_to_graph_map, num_graphs, emb_table, W_msg, W_self):
    ids = jnp.zeros((NPAD,), jnp.int32).at[:N_NODES].set(
        node_label_ids.astype(jnp.int32))
    srcs, dsts = [], []
    iota = jnp.arange(EPAD, dtype=jnp.int32)
    pad_src = iota % N_NODES
    pad_dst = N_NODES + iota % (NPAD - N_NODES)
    for a in (adjacency_list_0, adjacency_list_1, adjacency_list_2):
        a = a.astype(jnp.int32)
        srcs.append(pad_src.at[:E_PER].set(a[:, 0]).reshape(NW, NCH, CH))
        dsts.append(pad_dst.at[:E_PER].set(a[:, 1]).reshape(NW, NCH, CH))
    zeros = jnp.zeros((ROWS_PER_TILE, D), jnp.float32)

    h = _emb_gather(emb_table, ids)
    for layer in range(L):
        S = _seg_sums(h, srcs[0], dsts[0], srcs[1], dsts[1], srcs[2], dsts[2],
                      zeros)
        h = _tc_layer(h, S, W_self[layer], W_msg[layer])
    return h[:N_NODES]


# speedup vs baseline: 8.6702x; 1.0093x over previous
import functools

import jax
import jax.numpy as jnp
from jax import lax
from jax.experimental import pallas as pl
from jax.experimental.pallas import tpu as pltpu
from jax.experimental.pallas import tpu_sc as plsc

N_NODES = 10000
D = 128
NUM_ET = 3
E_PER = 106667
L = 2

NW = 32
NPAD = 10240
ROWS_PER_W = NPAD // NW
ROWS_PER_TILE = NPAD // 16

CH = 128
HC = CH // 2
NCH = 27
EW = NCH * CH
EPAD = NW * EW
NB = 2

GCH = 80
NGCH = ROWS_PER_W // GCH

_mesh = plsc.VectorSubcoreMesh(core_axis_name="c", subcore_axis_name="s")


@functools.partial(
    pl.kernel,
    out_type=jax.ShapeDtypeStruct((NPAD, D), jnp.float32),
    mesh=_mesh,
    scratch_types=[
        pltpu.VMEM((GCH,), jnp.int32),
        pltpu.VMEM((GCH,), jnp.int32),
        pltpu.VMEM((GCH, D), jnp.float32),
        pltpu.VMEM((GCH, D), jnp.float32),
        pltpu.SemaphoreType.DMA,
        pltpu.SemaphoreType.DMA,
    ],
)
def _emb_gather(table_hbm, ids_hbm, out_hbm, idxA, idxB, rowsA, rowsB,
                semA, semB):
    c = lax.axis_index("c")
    s = lax.axis_index("s")
    wid = s * 2 + c
    base = wid * ROWS_PER_W
    idx = (idxA, idxB)
    rows = (rowsA, rowsB)
    sem = (semA, semB)
    for k in range(NGCH):
        b = k % 2
        off = base + k * GCH
        pltpu.sync_copy(ids_hbm.at[pl.ds(off, GCH)], idx[b])
        pltpu.async_copy(table_hbm.at[idx[b]], rows[b], sem[b])
        if k > 0:
            pb = (k - 1) % 2
            poff = base + (k - 1) * GCH
            pltpu.make_async_copy(
                table_hbm.at[pl.ds(0, GCH)], rows[pb], sem[pb]).wait()
            pltpu.sync_copy(rows[pb], out_hbm.at[pl.ds(poff, GCH)])
    lb = (NGCH - 1) % 2
    pltpu.make_async_copy(
        table_hbm.at[pl.ds(0, GCH)], rows[lb], sem[lb]).wait()
    pltpu.sync_copy(rows[lb], out_hbm.at[pl.ds(base + (NGCH - 1) * GCH, GCH)])


@functools.partial(
    pl.kernel,
    out_type=jax.ShapeDtypeStruct((2, NUM_ET, NPAD, D), jnp.float32),
    mesh=_mesh,
    scratch_types=(
        [pltpu.VMEM_SHARED((NPAD, D), jnp.float32)]
        + [pltpu.VMEM((NCH, CH), jnp.int32)] * 2
        + [pltpu.VMEM((HC,), jnp.int32)] * (4 * NB)
        + [pltpu.VMEM((CH, D), jnp.float32)] * NB
        + [pltpu.SemaphoreType.DMA] * (2 * NB)
    ),
)
def _seg_sums(h_hbm, src0, dst0, src1, dst1, src2, dst2, zeros_hbm, out_hbm,
              acc_sh, src_all, dst_all, *bufs):
    srcb = [(bufs[2 * b], bufs[2 * b + 1]) for b in range(NB)]
    dstb = [(bufs[2 * NB + 2 * b], bufs[2 * NB + 2 * b + 1]) for b in range(NB)]
    rows = list(bufs[4 * NB:5 * NB])
    gsem = list(bufs[5 * NB:6 * NB])
    ssem = list(bufs[6 * NB:7 * NB])

    c = lax.axis_index("c")
    s = lax.axis_index("s")
    wid = s * 2 + c
    tile_lo = s * ROWS_PER_TILE

    def copy_idx(half_bufs, src_ref, k):
        d0, d1 = half_bufs
        for j in range(HC // 16):
            d0[pl.ds(j * 16, 16)] = src_ref[k, pl.ds(j * 16, 16)]
            d1[pl.ds(j * 16, 16)] = src_ref[k, pl.ds(HC + j * 16, 16)]

    def wait_rows(sem, buf):
        pltpu.make_async_copy(h_hbm.at[pl.ds(0, CH)], buf, sem).wait()

    pending = [False] * NB

    def fire_gather(k):
        b = k % NB
        if pending[b]:
            wait_rows(ssem[b], rows[b])
            pending[b] = False
        copy_idx(srcb[b], src_all, k)
        copy_idx(dstb[b], dst_all, k)
        pltpu.async_copy(h_hbm.at[srcb[b][0]], rows[b].at[pl.ds(0, HC)],
                         gsem[b])
        pltpu.async_copy(h_hbm.at[srcb[b][1]], rows[b].at[pl.ds(HC, HC)],
                         gsem[b])

    def fire_scatter(k):
        b = k % NB
        wait_rows(gsem[b], rows[b])
        pltpu.async_copy(rows[b].at[pl.ds(0, HC)], acc_sh.at[dstb[b][0]],
                         ssem[b], add=True)
        pltpu.async_copy(rows[b].at[pl.ds(HC, HC)], acc_sh.at[dstb[b][1]],
                         ssem[b], add=True)
        pending[b] = True

    def stage(t):
        src_hbm, dst_hbm = edge_lists[t]
        pltpu.sync_copy(src_hbm.at[wid], src_all)
        pltpu.sync_copy(dst_hbm.at[wid], dst_all)

    def zero_slice():
        pltpu.sync_copy(zeros_hbm, acc_sh.at[pl.ds(tile_lo, ROWS_PER_TILE)])

    def flush(t):
        pltpu.sync_copy(
            acc_sh.at[pl.ds(tile_lo, ROWS_PER_TILE)],
            out_hbm.at[c, t, pl.ds(tile_lo, ROWS_PER_TILE)],
        )

    edge_lists = ((src0, dst0), (src1, dst1), (src2, dst2))
    stage(0)
    fire_gather(0)
    fire_gather(1)
    zero_slice()
    plsc.subcore_barrier()
    for t in range(NUM_ET):
        for k in range(2, NCH):
            fire_scatter(k - 2)
            fire_gather(k)
        fire_scatter(NCH - 2)
        fire_scatter(NCH - 1)
        for b in range(NB):
            if pending[b]:
                wait_rows(ssem[b], rows[b])
                pending[b] = False
        if t + 1 < NUM_ET:
            stage(t + 1)
            fire_gather(0)
            fire_gather(1)
        plsc.subcore_barrier()
        flush(t)
        if t + 1 < NUM_ET:
            zero_slice()
            plsc.subcore_barrier()


BM = 2048


def _layer_body(h_ref, s_ref, wself_ref, wmsg_ref, out_ref):
    acc = jnp.dot(h_ref[...], wself_ref[...], preferred_element_type=jnp.float32)
    for t in range(NUM_ET):
        st = s_ref[0, t] + s_ref[1, t]
        acc += jnp.dot(st, wmsg_ref[t], preferred_element_type=jnp.float32)
    out_ref[...] = jnp.maximum(acc, 0.0)


def _tc_layer(h, S, wself, wmsg):
    return pl.pallas_call(
        _layer_body,
        grid=(NPAD // BM,),
        in_specs=[
            pl.BlockSpec((BM, D), lambda i: (i, 0)),
            pl.BlockSpec((2, NUM_ET, BM, D), lambda i: (0, 0, i, 0)),
            pl.BlockSpec((D, D), lambda i: (0, 0)),
            pl.BlockSpec((NUM_ET, D, D), lambda i: (0, 0, 0)),
        ],
        out_specs=pl.BlockSpec((BM, D), lambda i: (i, 0)),
        out_shape=jax.ShapeDtypeStruct((NPAD, D), jnp.float32),
    )(h, S, wself, wmsg)


def kernel(node_label_ids, adjacency_list_0, adjacency_list_1, adjacency_list_2,
           node_to_graph_map, num_graphs, emb_table, W_msg, W_self):
    ids = jnp.zeros((NPAD,), jnp.int32).at[:N_NODES].set(
        node_label_ids.astype(jnp.int32))
    srcs, dsts = [], []
    iota = jnp.arange(EPAD, dtype=jnp.int32)
    pad_src = iota % N_NODES
    pad_dst = N_NODES + iota % (NPAD - N_NODES)
    for a in (adjacency_list_0, adjacency_list_1, adjacency_list_2):
        a = a.astype(jnp.int32)
        srcs.append(pad_src.at[:E_PER].set(a[:, 0]).reshape(NW, NCH, CH))
        dsts.append(pad_dst.at[:E_PER].set(a[:, 1]).reshape(NW, NCH, CH))
    zeros = jnp.zeros((ROWS_PER_TILE, D), jnp.float32)

    h = _emb_gather(emb_table, ids)
    for layer in range(L):
        S = _seg_sums(h, srcs[0], dsts[0], srcs[1], dsts[1], srcs[2], dsts[2],
                      zeros)
        h = _tc_layer(h, S, W_self[layer], W_msg[layer])
    return h[:N_NODES]
